# baseline (speedup 1.0000x reference)
import functools
import jax
import jax.numpy as jnp
from jax.experimental import pallas as pl
from jax.experimental.pallas import tpu as pltpu

N = 50000
E = 800000
T = 12
HID = 96
HEADS = 8
DH = 12
HOR = 12
WID = HID + HEADS

ROW_BLK = 2000
ECHUNK = 8000
NEB = E // ECHUNK


def _mm_bias_body(x_ref, w_ref, b_ref, o_ref):
    o_ref[...] = jnp.dot(x_ref[...], w_ref[...],
                         preferred_element_type=jnp.float32) + b_ref[...]


def _mm_bias(x, w, b2):
    n, k = x.shape
    m = w.shape[1]
    return pl.pallas_call(
        _mm_bias_body,
        grid=(n // ROW_BLK,),
        in_specs=[
            pl.BlockSpec((ROW_BLK, k), lambda i: (i, 0)),
            pl.BlockSpec((k, m), lambda i: (0, 0)),
            pl.BlockSpec((1, m), lambda i: (0, 0)),
        ],
        out_specs=pl.BlockSpec((ROW_BLK, m), lambda i: (i, 0)),
        out_shape=jax.ShapeDtypeStruct((n, m), jnp.float32),
    )(x, w, b2)


def _mm_body(x_ref, w_ref, o_ref):
    o_ref[...] = jnp.dot(x_ref[...], w_ref[...],
                         preferred_element_type=jnp.float32)


def _mm(x, w):
    n, k = x.shape
    m = w.shape[1]
    return pl.pallas_call(
        _mm_body,
        grid=(n // ROW_BLK,),
        in_specs=[
            pl.BlockSpec((ROW_BLK, k), lambda i: (i, 0)),
            pl.BlockSpec((k, m), lambda i: (0, 0)),
        ],
        out_specs=pl.BlockSpec((ROW_BLK, m), lambda i: (i, 0)),
        out_shape=jax.ShapeDtypeStruct((n, m), jnp.float32),
    )(x, w)


def _edge_body(src_ref, dst_ref, ew_ref, hh_ref, ps_ref, pd_ref, sel_ref,
               out_ref):
    @pl.when(pl.program_id(0) == 0)
    def _init():
        out_ref[...] = jnp.zeros(out_ref.shape, out_ref.dtype)

    ps = ps_ref[...]
    pd = pd_ref[...]
    sel = sel_ref[...]

    def body(i, _):
        s = src_ref[0, 0, i]
        d = dst_ref[0, 0, i]
        w = ew_ref[0, 0, i]
        hs = hh_ref[pl.ds(s, 1), :]
        hd = hh_ref[pl.ds(d, 1), :]
        e96 = (jnp.dot(hs, ps, preferred_element_type=jnp.float32)
               + jnp.dot(hd, pd, preferred_element_type=jnp.float32))
        ex96 = jnp.exp(jnp.where(e96 >= 0, e96, 0.2 * e96))
        ex8 = jnp.dot(ex96, sel, preferred_element_type=jnp.float32)
        upd = jnp.concatenate([(ex96 * w) * hs, ex8], axis=1)
        out_ref[pl.ds(d, 1), :] += upd
        return 0

    jax.lax.fori_loop(0, ECHUNK, body, 0)


def _edge(src3, dst3, ew3, hh, psrc, pdst, sel):
    return pl.pallas_call(
        _edge_body,
        grid=(NEB,),
        in_specs=[
            pl.BlockSpec((1, 1, ECHUNK), lambda i: (i, 0, 0),
                         memory_space=pltpu.SMEM),
            pl.BlockSpec((1, 1, ECHUNK), lambda i: (i, 0, 0),
                         memory_space=pltpu.SMEM),
            pl.BlockSpec((1, 1, ECHUNK), lambda i: (i, 0, 0),
                         memory_space=pltpu.SMEM),
            pl.BlockSpec((N, HID), lambda i: (0, 0)),
            pl.BlockSpec((HID, HID), lambda i: (0, 0)),
            pl.BlockSpec((HID, HID), lambda i: (0, 0)),
            pl.BlockSpec((HID, HEADS), lambda i: (0, 0)),
        ],
        out_specs=pl.BlockSpec((N, WID), lambda i: (0, 0)),
        out_shape=jax.ShapeDtypeStruct((N, WID), jnp.float32),
    )(src3, dst3, ew3, hh, psrc, pdst, sel)


def _post_body(acc_ref, hres_ref, ls_ref, lb_ref, r96_ref, o_ref, *, do_elu):
    acc = acc_ref[...]
    outu = acc[:, :HID]
    ssum96 = jnp.dot(acc[:, HID:], r96_ref[...],
                     preferred_element_type=jnp.float32)
    hc = outu / (ssum96 + 1e-16)
    mu = jnp.mean(hc, axis=1, keepdims=True)
    v = jnp.mean((hc - mu) ** 2, axis=1, keepdims=True)
    hc = (hc - mu) / jnp.sqrt(v + 1e-5) * ls_ref[...] + lb_ref[...]
    if do_elu:
        hc = jnp.where(hc > 0, hc, jnp.exp(jnp.minimum(hc, 0.0)) - 1.0)
    o_ref[...] = hc + hres_ref[...]


def _post(acc, hres, ls2, lb2, r96t, do_elu):
    return pl.pallas_call(
        functools.partial(_post_body, do_elu=do_elu),
        grid=(N // ROW_BLK,),
        in_specs=[
            pl.BlockSpec((ROW_BLK, WID), lambda i: (i, 0)),
            pl.BlockSpec((ROW_BLK, HID), lambda i: (i, 0)),
            pl.BlockSpec((1, HID), lambda i: (0, 0)),
            pl.BlockSpec((1, HID), lambda i: (0, 0)),
            pl.BlockSpec((HEADS, HID), lambda i: (0, 0)),
        ],
        out_specs=pl.BlockSpec((ROW_BLK, HID), lambda i: (i, 0)),
        out_shape=jax.ShapeDtypeStruct((N, HID), jnp.float32),
    )(acc, hres, ls2, lb2, r96t)


def kernel(x, edge_index, edge_weight, enc_W, enc_b, W0, asrc0, adst0,
           lns0, lnb0, W1, asrc1, adst1, lns1, lnb1, dec_W, dec_b):
    f32 = jnp.float32
    hin = jnp.transpose(x, (0, 2, 1, 3)).reshape(N, T)
    src3 = edge_index[0].reshape(NEB, 1, ECHUNK)
    dst3 = edge_index[1].reshape(NEB, 1, ECHUNK)
    ew3 = edge_weight.reshape(NEB, 1, ECHUNK)

    pe = jnp.sin(jnp.arange(T, dtype=f32))
    enc_b2 = (pe @ enc_W + enc_b).reshape(1, HID)

    eye8 = jnp.eye(HEADS, dtype=f32)
    r96 = jnp.repeat(eye8, DH, axis=0)
    sel = jnp.zeros((HID, HEADS), f32).at[jnp.arange(HEADS) * DH,
                                          jnp.arange(HEADS)].set(1.0)

    def pmat(a):
        amat = (a[:, :, None] * eye8[:, None, :]).reshape(HID, HEADS)
        return amat @ r96.T

    ps0, pd0 = pmat(asrc0), pmat(adst0)
    ps1, pd1 = pmat(asrc1), pmat(adst1)

    h = _mm_bias(hin, enc_W, enc_b2)

    for (W, psrc, pdst, ls, lb, do_elu) in (
            (W0, ps0, pd0, lns0, lnb0, True),
            (W1, ps1, pd1, lns1, lnb1, False)):
        hh = _mm(h, W)
        acc = _edge(src3, dst3, ew3, hh, psrc, pdst, sel)
        h = _post(acc, h, ls.reshape(1, HID), lb.reshape(1, HID),
                  r96.T, do_elu)

    pred = _mm_bias(h, dec_W, dec_b.reshape(1, HOR))
    return jnp.transpose(pred, (1, 0))[None, :, :, None]
